# speedup vs baseline: 3.4321x; 3.4321x over previous
import jax
import jax.numpy as jnp
from jax.experimental import pallas as pl
from jax.experimental.pallas import tpu as pltpu


def _dct2_ortho_mat(N):
    n = jnp.arange(N, dtype=jnp.float32)
    k = n[:, None]
    C = jnp.cos(jnp.pi * (2.0 * n + 1.0) * k / (2.0 * N))
    scale = jnp.where(k == 0, jnp.sqrt(1.0 / N), jnp.sqrt(2.0 / N))
    return (C * scale).astype(jnp.float32)


def _pool_matrix():
    C64 = _dct2_ortho_mat(64)
    C32 = _dct2_ortho_mat(32)
    return C32.T @ C64[:32, :]


def _spectral_pool_kernel(x_ref, m_ref, mt_ref, o_ref):
    x = x_ref[0]
    m = m_ref[...]
    mt = mt_ref[...]

    y = jnp.dot(m, x.reshape(64, 64 * 64),
                preferred_element_type=jnp.float32)
    y = y.reshape(32, 64, 64)

    y = y.transpose(0, 2, 1)
    y = jnp.dot(y.reshape(32 * 64, 64), mt,
                preferred_element_type=jnp.float32)
    y = y.reshape(32, 64, 32)

    y = y.transpose(0, 2, 1)
    y = jnp.dot(y.reshape(32 * 32, 64), mt,
                preferred_element_type=jnp.float32)
    o_ref[0] = y.reshape(32, 32, 32)


def kernel(x):
    B, C, D, H, W = x.shape
    M = _pool_matrix()
    xr = x.reshape(B * C, D, H, W)

    out = pl.pallas_call(
        _spectral_pool_kernel,
        grid=(B * C,),
        in_specs=[
            pl.BlockSpec((1, D, H, W), lambda i: (i, 0, 0, 0)),
            pl.BlockSpec((32, 64), lambda i: (0, 0)),
            pl.BlockSpec((64, 32), lambda i: (0, 0)),
        ],
        out_specs=pl.BlockSpec((1, 32, 32, 32), lambda i: (i, 0, 0, 0)),
        out_shape=jax.ShapeDtypeStruct((B * C, 32, 32, 32), jnp.float32),
        compiler_params=pltpu.CompilerParams(
            dimension_semantics=("parallel",),
        ),
    )(xr, M, M.T)

    return out.reshape(B, C, 32, 32, 32)


# speedup vs baseline: 3.4659x; 1.0099x over previous
import jax
import jax.numpy as jnp
from jax.experimental import pallas as pl
from jax.experimental.pallas import tpu as pltpu


def _dct2_ortho_mat(N):
    n = jnp.arange(N, dtype=jnp.float32)
    k = n[:, None]
    C = jnp.cos(jnp.pi * (2.0 * n + 1.0) * k / (2.0 * N))
    scale = jnp.where(k == 0, jnp.sqrt(1.0 / N), jnp.sqrt(2.0 / N))
    return (C * scale).astype(jnp.float32)


def _pool_matrix():
    C64 = _dct2_ortho_mat(64)
    C32 = _dct2_ortho_mat(32)
    return C32.T @ C64[:32, :]


_NSLICE = 2


def _spectral_pool_kernel(x_ref, m_ref, mt_ref, o_ref):
    m = m_ref[...]
    mt = mt_ref[...]

    for s in range(_NSLICE):
        x = x_ref[s]
        t = jnp.dot(x.reshape(64 * 64, 64), mt,
                    preferred_element_type=jnp.float32)
        t = t.reshape(64, 64, 32).transpose(0, 2, 1)
        t = jnp.dot(t.reshape(64 * 32, 64), mt,
                    preferred_element_type=jnp.float32)
        t = t.reshape(64, 32, 32).transpose(0, 2, 1)
        t = t.reshape(64, 32 * 32)
        o_ref[s] = jnp.dot(m, t,
                           preferred_element_type=jnp.float32)


def kernel(x):
    B, C, D, H, W = x.shape
    M = _pool_matrix()
    xr = x.reshape(B * C, D, H, W)
    n = B * C // _NSLICE

    out = pl.pallas_call(
        _spectral_pool_kernel,
        grid=(n,),
        in_specs=[
            pl.BlockSpec((_NSLICE, D, H, W), lambda i: (i, 0, 0, 0)),
            pl.BlockSpec((32, 64), lambda i: (0, 0)),
            pl.BlockSpec((64, 32), lambda i: (0, 0)),
        ],
        out_specs=pl.BlockSpec((_NSLICE, 32, 32 * 32), lambda i: (i, 0, 0)),
        out_shape=jax.ShapeDtypeStruct((B * C, 32, 32 * 32), jnp.float32),
        compiler_params=pltpu.CompilerParams(
            dimension_semantics=("parallel",),
        ),
    )(xr, M, M.T)

    return out.reshape(B, C, 32, 32, 32)
